# baseline (speedup 1.0000x reference)
import functools

import jax
import jax.numpy as jnp
from jax import lax
from jax.experimental import pallas as pl
from jax.experimental.pallas import tpu as pltpu
from jax.experimental.pallas import tpu_sc as plsc

B, T, D = 16, 2048, 256
VOCAB = 100000

_NC = 2
_NS = 16
_NW = _NC * _NS
_TOK_W = B * T // _NW
_SUB = 8
_CT = _TOK_W // _SUB
_DV = D // 16


def _sc_dense(vs3, lens, idx3, weights):
    mesh = plsc.VectorSubcoreMesh(core_axis_name="c", subcore_axis_name="s")

    @functools.partial(
        pl.kernel,
        mesh=mesh,
        out_type=jax.ShapeDtypeStruct((2, 2, B, D), jnp.float32),
        scratch_types=[
            pltpu.VMEM((2, _CT, D), jnp.float32),
            pltpu.VMEM((_SUB, _CT), jnp.int32),
            pltpu.VMEM((_SUB, _CT), jnp.float32),
            pltpu.VMEM((B, 16), jnp.int32),
            pltpu.VMEM((2, D), jnp.float32),
            pltpu.SemaphoreType.DMA,
            pltpu.SemaphoreType.DMA,
            pltpu.SemaphoreType.DMA,
        ],
        compiler_params=pltpu.CompilerParams(needs_layout_passes=False),
    )
    def dense_kernel(vs_hbm, lens_hbm, idx_hbm, w_hbm, out_hbm,
                     buf_v, idx_v, w_v, lens_v, out_v, sem_g, sem_a, sem_b):
        wid = lax.axis_index("s") * _NC + lax.axis_index("c")
        bb = wid // 2
        h = wid % 2
        t0 = h * _TOK_W

        pltpu.sync_copy(lens_hbm, lens_v)
        pltpu.sync_copy(idx_hbm.at[wid], idx_v)
        sems = (sem_a, sem_b)
        first = pltpu.async_copy(vs_hbm.at[wid, pl.ds(0, _CT)], buf_v.at[0],
                                 sems[0])
        gathers = [
            pltpu.async_copy(w_hbm.at[idx_v.at[j]], w_v.at[j], sem_g)
            for j in range(_SUB)
        ]
        len_row = lens_v[bb, pl.ds(0, 16)]

        accs = tuple(jnp.zeros((16,), jnp.float32) for _ in range(2 * _DV))
        pending = first
        for g in gathers:
            g.wait()
        lane = lax.iota(jnp.int32, 16)
        for c in range(_SUB):
            if c + 1 < _SUB:
                nxt = pltpu.async_copy(
                    vs_hbm.at[wid, pl.ds((c + 1) * _CT, _CT)],
                    buf_v.at[(c + 1) % 2], sems[(c + 1) % 2])
            pending.wait()
            if c + 1 < _SUB:
                pending = nxt
            buf = buf_v.at[c % 2]

            def grp_body(g, carry, c=c, buf=buf):
                w16 = w_v[c, pl.ds(g * 16, 16)]
                tg = t0 + c * _CT + g * 16
                valid = (lane + tg) < len_row
                wm16 = jnp.where(valid, w16, 0.0)
                mm16 = jnp.where(valid, 1.0, 0.0)
                cur = list(carry)
                for k in range(16):
                    t = g * 16 + k
                    mm = mm16[k]
                    wm = wm16[k]
                    for j in range(_DV):
                        v = buf[t, pl.ds(j * 16, 16)]
                        cur[j] = cur[j] + mm * v
                        cur[_DV + j] = cur[_DV + j] + wm * v
                return tuple(cur)

            accs = lax.fori_loop(0, _CT // 16, grp_body, accs)

        for j in range(_DV):
            out_v[0, pl.ds(j * 16, 16)] = accs[j]
            out_v[1, pl.ds(j * 16, 16)] = accs[_DV + j]
        pltpu.sync_copy(out_v.at[0], out_hbm.at[0, h, bb])
        pltpu.sync_copy(out_v.at[1], out_hbm.at[1, h, bb])

    return dense_kernel(vs3, lens, idx3, weights)


def _combine_body(p_ref, y_ref, yh_ref):
    s = p_ref[0, 0] + p_ref[0, 1]
    yh = p_ref[1, 0] + p_ref[1, 1]
    denom = jnp.sqrt(jnp.sum(jnp.abs(s), axis=1, keepdims=True))
    y_ref[...] = s / denom
    yh_ref[...] = yh


def kernel(vector_sequence, sentence_length, word_sequence, weights):
    vs3 = vector_sequence.reshape(_NW, _TOK_W, D)
    idx3 = word_sequence.astype(jnp.int32).reshape(_NW, _SUB, _CT)
    lens = jnp.broadcast_to(
        sentence_length.astype(jnp.int32)[:, None], (B, 16))
    parts = _sc_dense(vs3, lens, idx3, weights)
    y, y_hat = pl.pallas_call(
        _combine_body,
        out_shape=[
            jax.ShapeDtypeStruct((B, D), jnp.float32),
            jax.ShapeDtypeStruct((B, D), jnp.float32),
        ],
    )(parts)
    return y, y_hat


# speedup vs baseline: 1.0249x; 1.0249x over previous
import functools

import jax
import jax.numpy as jnp
from jax import lax
from jax.experimental import pallas as pl
from jax.experimental.pallas import tpu as pltpu
from jax.experimental.pallas import tpu_sc as plsc

B, T, D = 16, 2048, 256
VOCAB = 100000

_NC = 2
_NS = 16
_NW = _NC * _NS
_N_IDX = B * T
_CHUNK = _N_IDX // _NW
_SUB = 8
_SUBW = _CHUNK // _SUB


def _sc_gather(weights, idx3):
    mesh = plsc.VectorSubcoreMesh(core_axis_name="c", subcore_axis_name="s")
    nw, sub, subw = idx3.shape

    @functools.partial(
        pl.kernel,
        mesh=mesh,
        out_type=jax.ShapeDtypeStruct((nw, sub, subw), jnp.float32),
        scratch_types=[
            pltpu.VMEM((sub, subw), jnp.int32),
            pltpu.VMEM((sub, subw), jnp.float32),
            pltpu.SemaphoreType.DMA,
        ],
        compiler_params=pltpu.CompilerParams(needs_layout_passes=False),
    )
    def gather_kernel(w_hbm, idx_hbm, out_hbm, idx_v, rows_v, sem):
        wid = lax.axis_index("s") * _NC + lax.axis_index("c")
        pltpu.sync_copy(idx_hbm.at[wid], idx_v)
        copies = [
            pltpu.async_copy(w_hbm.at[idx_v.at[j]], rows_v.at[j], sem)
            for j in range(sub)
        ]
        for c in copies:
            c.wait()
        pltpu.sync_copy(rows_v, out_hbm.at[wid])

    return gather_kernel(weights, idx3)


_NROW = 8
_GB = B // _NROW


def _one_row(length, vs, w_row_raw, y_ref, yh_ref):
    pos = lax.broadcasted_iota(jnp.int32, (1, T), 1)
    maskf = (pos < length).astype(jnp.float32)
    w_row = w_row_raw * maskf
    a = jnp.concatenate([maskf, w_row], axis=0)
    acc = jnp.dot(a, vs, preferred_element_type=jnp.float32)
    s = acc[0:1, :]
    denom = jnp.sqrt(jnp.sum(jnp.abs(s)))
    y_ref[0, :, :] = s / denom
    yh_ref[0, :, :] = acc[1:2, :]


def _tc_body(len_ref, *refs):
    vs_refs = refs[:_NROW]
    w_refs = refs[_NROW:2 * _NROW]
    y_refs = refs[2 * _NROW:3 * _NROW]
    yh_refs = refs[3 * _NROW:]
    b = pl.program_id(0)
    for k in range(_NROW):
        _one_row(len_ref[b + k * _GB], vs_refs[k][0], w_refs[k][0],
                 y_refs[k], yh_refs[k])


def kernel(vector_sequence, sentence_length, word_sequence, weights):
    idx3 = word_sequence.astype(jnp.int32).reshape(_NW, _SUB, _SUBW)
    w_tok = _sc_gather(weights, idx3)
    w3 = w_tok.reshape(B, 1, T)
    lens = sentence_length.astype(jnp.int32)

    def _off(k):
        return lambda b: (b + k * _GB, 0, 0)

    vs_specs = [pl.BlockSpec((1, T, D), _off(k)) for k in range(_NROW)]
    w_specs = [pl.BlockSpec((1, 1, T), _off(k)) for k in range(_NROW)]
    out_spec = pl.BlockSpec((1, 1, D), lambda b: (b, 0, 0))
    out_ty = jax.ShapeDtypeStruct((_GB, 1, D), jnp.float32)
    outs = pl.pallas_call(
        _tc_body,
        grid=(_GB,),
        in_specs=[
            pl.BlockSpec(memory_space=pltpu.SMEM),
            *vs_specs,
            *w_specs,
        ],
        out_specs=[out_spec] * (2 * _NROW),
        out_shape=[out_ty] * (2 * _NROW),
    )(lens, *([vector_sequence] * _NROW), *([w3] * _NROW))
    y = jnp.concatenate(outs[:_NROW], axis=0).reshape(B, D)
    y_hat = jnp.concatenate(outs[_NROW:], axis=0).reshape(B, D)
    return y, y_hat


# speedup vs baseline: 1.2936x; 1.2622x over previous
import functools

import jax
import jax.numpy as jnp
from jax import lax
from jax.experimental import pallas as pl
from jax.experimental.pallas import tpu as pltpu
from jax.experimental.pallas import tpu_sc as plsc

B, T, D = 16, 2048, 256
VOCAB = 100000

_NC = 2
_NS = 16
_NW = _NC * _NS
_N_IDX = B * T
_CHUNK = _N_IDX // _NW
_SUB = 8
_SUBW = _CHUNK // _SUB


def _sc_gather(weights, idx3):
    mesh = plsc.VectorSubcoreMesh(core_axis_name="c", subcore_axis_name="s")
    nw, sub, subw = idx3.shape

    @functools.partial(
        pl.kernel,
        mesh=mesh,
        out_type=jax.ShapeDtypeStruct((nw, sub, subw), jnp.float32),
        scratch_types=[
            pltpu.VMEM((sub, subw), jnp.int32),
            pltpu.VMEM((sub, subw), jnp.float32),
            pltpu.SemaphoreType.DMA,
        ],
        compiler_params=pltpu.CompilerParams(needs_layout_passes=False),
    )
    def gather_kernel(w_hbm, idx_hbm, out_hbm, idx_v, rows_v, sem):
        wid = lax.axis_index("s") * _NC + lax.axis_index("c")
        pltpu.sync_copy(idx_hbm.at[wid], idx_v)
        copies = [
            pltpu.async_copy(w_hbm.at[idx_v.at[j]], rows_v.at[j], sem)
            for j in range(sub)
        ]
        for c in copies:
            c.wait()
        pltpu.sync_copy(rows_v, out_hbm.at[wid])

    return gather_kernel(weights, idx3)


_NROW = 4
_GB = B // _NROW


def _one_row(length, vs, w_row_raw, y_ref, yh_ref):
    pos = lax.broadcasted_iota(jnp.int32, (1, T), 1)
    maskf = (pos < length).astype(jnp.float32)
    w_row = w_row_raw * maskf
    a = jnp.concatenate([maskf, w_row], axis=0)
    acc = jnp.dot(a, vs, preferred_element_type=jnp.float32)
    s = acc[0:1, :]
    denom = jnp.sqrt(jnp.sum(jnp.abs(s)))
    y_ref[0, :, :] = s / denom
    yh_ref[0, :, :] = acc[1:2, :]


def _tc_body(len_ref, *refs):
    vs_refs = refs[:_NROW]
    w_refs = refs[_NROW:2 * _NROW]
    y_refs = refs[2 * _NROW:3 * _NROW]
    yh_refs = refs[3 * _NROW:]
    b = pl.program_id(0)
    for k in range(_NROW):
        _one_row(len_ref[b + k * _GB], vs_refs[k][0], w_refs[k][0],
                 y_refs[k], yh_refs[k])


def kernel(vector_sequence, sentence_length, word_sequence, weights):
    idx3 = word_sequence.astype(jnp.int32).reshape(_NW, _SUB, _SUBW)
    w_tok = _sc_gather(weights, idx3)
    w3 = w_tok.reshape(B, 1, T)
    lens = sentence_length.astype(jnp.int32)

    def _off(k):
        return lambda b: (b + k * _GB, 0, 0)

    vs_specs = [pl.BlockSpec((1, T, D), _off(k)) for k in range(_NROW)]
    w_specs = [pl.BlockSpec((1, 1, T), _off(k)) for k in range(_NROW)]
    out_spec = pl.BlockSpec((1, 1, D), lambda b: (b, 0, 0))
    out_ty = jax.ShapeDtypeStruct((_GB, 1, D), jnp.float32)
    outs = pl.pallas_call(
        _tc_body,
        grid=(_GB,),
        in_specs=[
            pl.BlockSpec(memory_space=pltpu.SMEM),
            *vs_specs,
            *w_specs,
        ],
        out_specs=[out_spec] * (2 * _NROW),
        out_shape=[out_ty] * (2 * _NROW),
    )(lens, *([vector_sequence] * _NROW), *([w3] * _NROW))
    y = jnp.concatenate(outs[:_NROW], axis=0).reshape(B, D)
    y_hat = jnp.concatenate(outs[_NROW:], axis=0).reshape(B, D)
    return y, y_hat
